# baseline (speedup 1.0000x reference)
import functools

import jax
import jax.numpy as jnp
from jax import lax
from jax.experimental import pallas as pl
from jax.experimental.pallas import tpu as pltpu
from jax.experimental.pallas import tpu_sc as plsc

B = 16384
GMF_DIM = 16
MLP_DIM = 64

_NC = 2
_NS = 16
_NW = _NC * _NS
_BPW = B // _NW


def _sc_gather_body(user_hbm, item_hbm, gu_tab, gi_tab, mu_tab, mi_tab,
                    gu_out, gi_out, mu_out, mi_out,
                    idx_u, idx_i, gu_v, gi_v, mu_v, mi_v,
                    s0, s1, s2, s3):
    wid = lax.axis_index("s") * _NC + lax.axis_index("c")
    base = wid * _BPW
    pltpu.sync_copy(user_hbm.at[pl.ds(base, _BPW)], idx_u)
    pltpu.sync_copy(item_hbm.at[pl.ds(base, _BPW)], idx_i)
    c0 = pltpu.async_copy(gu_tab.at[idx_u], gu_v, s0)
    c1 = pltpu.async_copy(gi_tab.at[idx_i], gi_v, s1)
    c2 = pltpu.async_copy(mu_tab.at[idx_u], mu_v, s2)
    c3 = pltpu.async_copy(mi_tab.at[idx_i], mi_v, s3)
    c2.wait()
    pltpu.sync_copy(mu_v, mu_out.at[pl.ds(base, _BPW)])
    c3.wait()
    pltpu.sync_copy(mi_v, mi_out.at[pl.ds(base, _BPW)])
    c0.wait()
    pltpu.sync_copy(gu_v, gu_out.at[pl.ds(base, _BPW)])
    c1.wait()
    pltpu.sync_copy(gi_v, gi_out.at[pl.ds(base, _BPW)])


def _sc_gather(user, item, gu_tab, gi_tab, mu_tab, mi_tab):
    mesh = plsc.VectorSubcoreMesh(core_axis_name="c", subcore_axis_name="s")
    f32 = jnp.float32
    out_type = (
        jax.ShapeDtypeStruct((B, GMF_DIM), f32),
        jax.ShapeDtypeStruct((B, GMF_DIM), f32),
        jax.ShapeDtypeStruct((B, MLP_DIM), f32),
        jax.ShapeDtypeStruct((B, MLP_DIM), f32),
    )
    scratch = [
        pltpu.VMEM((_BPW,), jnp.int32),
        pltpu.VMEM((_BPW,), jnp.int32),
        pltpu.VMEM((_BPW, GMF_DIM), f32),
        pltpu.VMEM((_BPW, GMF_DIM), f32),
        pltpu.VMEM((_BPW, MLP_DIM), f32),
        pltpu.VMEM((_BPW, MLP_DIM), f32),
        pltpu.SemaphoreType.DMA,
        pltpu.SemaphoreType.DMA,
        pltpu.SemaphoreType.DMA,
        pltpu.SemaphoreType.DMA,
    ]
    fn = pl.kernel(_sc_gather_body, out_type=out_type, mesh=mesh,
                   scratch_types=scratch,
                   compiler_params=pltpu.CompilerParams(
                       use_tc_tiling_on_sc=False))
    return fn(user, item, gu_tab, gi_tab, mu_tab, mi_tab)


def _tc_mlp_body(gu_ref, gi_ref, mu_ref, mi_ref, w1_ref, w2_ref, wf_ref,
                 out_ref):
    dn = (((1,), (1,)), ((), ()))
    w1 = w1_ref[...]
    h1 = lax.dot_general(mu_ref[...], w1[:, :MLP_DIM], dn,
                         preferred_element_type=jnp.float32)
    h1 = h1 + lax.dot_general(mi_ref[...], w1[:, MLP_DIM:], dn,
                              preferred_element_type=jnp.float32)
    h1 = jnp.maximum(h1, 0.0)
    h2 = lax.dot_general(h1, w2_ref[...], dn,
                         preferred_element_type=jnp.float32)
    h2 = jnp.maximum(h2, 0.0)
    gmf_x = gu_ref[...] * gi_ref[...]
    wf = wf_ref[...]
    logit = lax.dot_general(gmf_x, wf[:, :GMF_DIM], dn,
                            preferred_element_type=jnp.float32)
    logit = logit + lax.dot_general(h2, wf[:, GMF_DIM:], dn,
                                    preferred_element_type=jnp.float32)
    out_ref[...] = jax.nn.sigmoid(logit)


def _tc_mlp(gu, gi, mu, mi, W1, W2, Wf):
    blk = 2048
    grid = (B // blk,)
    f32 = jnp.float32
    full = lambda shape: pl.BlockSpec(shape, lambda i: (0, 0))
    return pl.pallas_call(
        _tc_mlp_body,
        grid=grid,
        in_specs=[
            pl.BlockSpec((blk, GMF_DIM), lambda i: (i, 0)),
            pl.BlockSpec((blk, GMF_DIM), lambda i: (i, 0)),
            pl.BlockSpec((blk, MLP_DIM), lambda i: (i, 0)),
            pl.BlockSpec((blk, MLP_DIM), lambda i: (i, 0)),
            full(W1.shape),
            full(W2.shape),
            full(Wf.shape),
        ],
        out_specs=pl.BlockSpec((blk, 1), lambda i: (i, 0)),
        out_shape=jax.ShapeDtypeStruct((B, 1), f32),
    )(gu, gi, mu, mi, W1, W2, Wf)


def kernel(x, gmf_user_table, gmf_item_table, mlp_user_table,
           mlp_item_table, W1, W2, Wf):
    user = x[:, 0].astype(jnp.int32)
    item = x[:, 1].astype(jnp.int32)
    gu, gi, mu, mi = _sc_gather(user, item, gmf_user_table, gmf_item_table,
                                mlp_user_table, mlp_item_table)
    return _tc_mlp(gu, gi, mu, mi, W1, W2, Wf)
